# baseline (speedup 1.0000x reference)
import functools

import jax
import jax.numpy as jnp
from jax import lax
from jax.experimental import pallas as pl
from jax.experimental.pallas import tpu as pltpu
from jax.experimental.pallas import tpu_sc as plsc

N_LAYERS = 100000
CODE_DIM = 64
BATCH = 16384

NC = 2
NS = 16
NW = NC * NS
B_PER_W = BATCH // NW
CHUNK = 128
N_CHUNKS = B_PER_W // CHUNK

_mesh = plsc.VectorSubcoreMesh(core_axis_name="c", subcore_axis_name="s")


@functools.partial(
    pl.kernel,
    mesh=_mesh,
    out_type=jax.ShapeDtypeStruct((BATCH, CODE_DIM), jnp.float32),
    scratch_types=[
        pltpu.VMEM((B_PER_W,), jnp.int32),
        pltpu.VMEM((B_PER_W, CODE_DIM), jnp.float32),
        pltpu.SemaphoreType.DMA((N_CHUNKS,)),
        pltpu.SemaphoreType.DMA,
    ],
)
def _gather_kernel(codes_hbm, idx_hbm, out_hbm, idx_v, rows_v, gsem, wsem):
    wid = lax.axis_index("s") * NC + lax.axis_index("c")
    base = wid * B_PER_W
    pltpu.sync_copy(idx_hbm.at[pl.ds(base, B_PER_W)], idx_v)

    def fire_chunk(c):
        def fire(g, _):
            j = c * CHUNK + g * 16
            v = idx_v[pl.ds(j, 16)]
            for k in range(16):
                pltpu.async_copy(
                    codes_hbm.at[v[k]], rows_v.at[j + k], gsem.at[c]
                )
            return _

        lax.fori_loop(0, CHUNK // 16, fire, 0)

    def drain_and_write(c):
        chunk = rows_v.at[pl.ds(c * CHUNK, CHUNK)]
        pltpu.make_async_copy(
            codes_hbm.at[pl.ds(0, CHUNK)], chunk, gsem.at[c]
        ).wait()
        pltpu.async_copy(
            chunk, out_hbm.at[pl.ds(base + c * CHUNK, CHUNK)], wsem
        )

    for c in range(N_CHUNKS):
        fire_chunk(c)
        if c > 0:
            drain_and_write(c - 1)
    drain_and_write(N_CHUNKS - 1)
    pltpu.make_async_copy(
        codes_hbm.at[pl.ds(0, B_PER_W)], rows_v, wsem
    ).wait()


def kernel(layer_idx, codes):
    return _gather_kernel(codes, layer_idx)


# speedup vs baseline: 1.0001x; 1.0001x over previous
import functools

import jax
import jax.numpy as jnp
from jax import lax
from jax.experimental import pallas as pl
from jax.experimental.pallas import tpu as pltpu
from jax.experimental.pallas import tpu_sc as plsc

N_LAYERS = 100000
CODE_DIM = 64
BATCH = 16384

NC = 2
NS = 16
NW = NC * NS
B_PER_W = BATCH // NW

_mesh = plsc.VectorSubcoreMesh(core_axis_name="c", subcore_axis_name="s")


@functools.partial(
    pl.kernel,
    mesh=_mesh,
    out_type=jax.ShapeDtypeStruct((BATCH, CODE_DIM), jnp.float32),
    scratch_types=[
        pltpu.VMEM((B_PER_W,), jnp.int32),
        pltpu.VMEM((B_PER_W, CODE_DIM), jnp.float32),
        pltpu.SemaphoreType.DMA,
    ],
    compiler_params=pltpu.CompilerParams(skip_device_barrier=True),
)
def _gather_kernel(codes_hbm, idx_hbm, out_hbm, idx_v, rows_v, sem):
    wid = lax.axis_index("s") * NC + lax.axis_index("c")
    base = wid * B_PER_W
    pltpu.sync_copy(idx_hbm.at[pl.ds(base, B_PER_W)], idx_v)

    def fire(g, _):
        j = g * 64
        vs = [idx_v[pl.ds(j + 16 * m, 16)] for m in range(4)]
        for m in range(4):
            for k in range(16):
                pltpu.async_copy(
                    codes_hbm.at[vs[m][k]], rows_v.at[j + 16 * m + k], sem
                )
        return _

    lax.fori_loop(0, B_PER_W // 64, fire, 0)
    pltpu.make_async_copy(codes_hbm.at[pl.ds(0, B_PER_W)], rows_v, sem).wait()
    pltpu.sync_copy(rows_v, out_hbm.at[pl.ds(base, B_PER_W)])


def kernel(layer_idx, codes):
    return _gather_kernel(codes, layer_idx)
